# Initial kernel scaffold; baseline (speedup 1.0000x reference)
import jax
import jax.numpy as jnp
from jax.experimental import pallas as pl


def kernel(lhs_x, lhs_edge_index, lhs_batch, rhs_x, rhs_edge_index, rhs_batch, emb, W1, b1, W2, b2, Wout, bout):
    raise NotImplementedError("write your pallas kernel here")


# speedup vs baseline: 32.1111x; 32.1111x over previous
import functools

import jax
import jax.numpy as jnp
from jax import lax
from jax.experimental import pallas as pl
from jax.experimental.pallas import tpu as pltpu
from jax.experimental.pallas import tpu_sc as plsc

N = 100000
E = 1600000
G = 128
EMB = 16
HID = 32

NC = 2
NS = 16
NW = NC * NS

DEG_EPT = 100352
DEG_BLOCKS = 98
SC_EPT = 50176
SC_BLOCKS = 49
ACC_ROWS = 100352
NB = 2000
NBLK = N // NB


def _deg_body(dst_ref, out_ref, acc, dbuf, sem):
    c = lax.axis_index("c")
    s = lax.axis_index("s")
    w = c * NS + s
    z16 = jnp.zeros((16,), jnp.float32)
    ones16 = jnp.ones((16,), jnp.float32)

    def zero_body(i, _):
        acc[pl.ds(i * 16, 16)] = z16
        return 0

    lax.fori_loop(0, (N + 16) // 16, zero_body, 0)

    def blk_body(b, _):
        pltpu.sync_copy(dst_ref.at[w, b], dbuf)
        for k in range(64):
            idx = dbuf[pl.ds(k * 16, 16)]
            plsc.addupdate_scatter(acc, [idx], ones16)
        return 0

    lax.fori_loop(0, DEG_BLOCKS, blk_body, 0)

    def wb_body(i, _):
        pltpu.sync_copy(acc.at[pl.ds(i * NB, NB)], out_ref.at[i, w])
        return 0

    lax.fori_loop(0, NBLK, wb_body, 0)


_SC_PARAMS = pltpu.CompilerParams(
    needs_layout_passes=False, use_tc_tiling_on_sc=False
)

@functools.cache
def _deg_kernel_fn():
    return pl.kernel(
        _deg_body,
        out_type=jax.ShapeDtypeStruct((NBLK, NW, NB), jnp.float32),
        compiler_params=_SC_PARAMS,
        mesh=plsc.VectorSubcoreMesh(
            core_axis_name="c", subcore_axis_name="s",
            num_cores=NC, num_subcores=NS,
        ),
        scratch_types=[
            pltpu.VMEM((N + 16,), jnp.float32),
            pltpu.VMEM((1024,), jnp.int32),
            pltpu.SemaphoreType.DMA,
        ],
    )


def _deg_kernel(deg_in):
    return _deg_kernel_fn()(deg_in)


def _scat_body(vtab_ref, src_ref, dst_ref, out_ref, acc, sbuf, dbuf, rows, sem):
    c = lax.axis_index("c")
    s = lax.axis_index("s")
    w = c * NS + s
    z16 = jnp.zeros((16,), jnp.float32)

    def zrow(i, _):
        rows[i, :] = z16
        return 0

    lax.fori_loop(0, 1024, zrow, 0)
    for j in range(7):
        ch = s + NS * j

        @pl.when(ch < DEG_BLOCKS)
        def _():
            pltpu.sync_copy(rows, acc.at[pl.ds(ch * 1024, 1024)])

    plsc.subcore_barrier()

    def blk_body(b, _):
        pltpu.sync_copy(src_ref.at[w, b], sbuf)
        pltpu.sync_copy(dst_ref.at[w, b], dbuf)
        descs = []
        for j in range(8):
            descs.append(
                pltpu.async_copy(
                    vtab_ref.at[sbuf.at[j]], rows.at[pl.ds(j * 128, 128)], sem
                )
            )
        for d in descs:
            d.wait()
        for j in range(8):
            pltpu.sync_copy(
                rows.at[pl.ds(j * 128, 128)], acc.at[dbuf.at[j]], add=True
            )
        return 0

    lax.fori_loop(0, SC_BLOCKS, blk_body, 0)
    plsc.subcore_barrier()
    rpt = N // NS
    pltpu.sync_copy(acc.at[pl.ds(s * rpt, rpt)], out_ref.at[c, pl.ds(s * rpt, rpt)])


@functools.cache
def _scat_kernel_fn():
    return pl.kernel(
        _scat_body,
        out_type=jax.ShapeDtypeStruct((NC, N, 16), jnp.float32),
        compiler_params=_SC_PARAMS,
        mesh=plsc.VectorSubcoreMesh(
            core_axis_name="c", subcore_axis_name="s",
            num_cores=NC, num_subcores=NS,
        ),
        scratch_types=[
            pltpu.VMEM_SHARED((ACC_ROWS, 16), jnp.float32),
            pltpu.VMEM((8, 128), jnp.int32),
            pltpu.VMEM((8, 128), jnp.int32),
            pltpu.VMEM((1024, 16), jnp.float32),
            pltpu.SemaphoreType.DMA,
        ],
    )


def _scat_kernel(vtab, src3, dst3):
    return _scat_kernel_fn()(vtab, src3, dst3)


def _prep_body(degp_ref, x_ref, batch_ref, emb_ref, dinv_ref, g1_ref, cnt_ref,
               *, lo):
    i = pl.program_id(0)
    deg = jnp.sum(degp_ref[0, lo:lo + NS, :], axis=0) + 1.0
    dinv = lax.rsqrt(jnp.maximum(deg, 1.0))
    x = x_ref[...]
    h0 = jnp.zeros((NB, EMB), jnp.float32)
    for k in range(11):
        h0 = h0 + jnp.where(x == k, 1.0, 0.0) * emb_ref[k, :][None, :]
    dinv_ref[...] = dinv[:, None]
    g1_ref[...] = dinv[:, None] * h0
    onehot = jnp.where(
        batch_ref[...] == lax.broadcasted_iota(jnp.int32, (NB, G), 1), 1.0, 0.0
    )

    @pl.when(i == 0)
    def _():
        cnt_ref[...] = jnp.zeros((1, G), jnp.float32)

    cnt_ref[...] += jnp.sum(onehot, axis=0, keepdims=True)


def _prep(degp, x, batch2d, embp, lo):
    return pl.pallas_call(
        functools.partial(_prep_body, lo=lo),
        grid=(NBLK,),
        in_specs=[
            pl.BlockSpec((1, NW, NB), lambda i: (i, 0, 0)),
            pl.BlockSpec((NB, 1), lambda i: (i, 0)),
            pl.BlockSpec((NB, 1), lambda i: (i, 0)),
            pl.BlockSpec((16, EMB), lambda i: (0, 0)),
        ],
        out_specs=[
            pl.BlockSpec((NB, 1), lambda i: (i, 0)),
            pl.BlockSpec((NB, EMB), lambda i: (i, 0)),
            pl.BlockSpec((1, G), lambda i: (0, 0)),
        ],
        out_shape=[
            jax.ShapeDtypeStruct((N, 1), jnp.float32),
            jax.ShapeDtypeStruct((N, EMB), jnp.float32),
            jax.ShapeDtypeStruct((1, G), jnp.float32),
        ],
    )(degp, x, batch2d, embp)


def _mid_body(tp_ref, g1_ref, dinv_ref, W1_ref, b1_ref, glo_ref, ghi_ref):
    t = tp_ref[0] + tp_ref[1] + g1_ref[...]
    a = dinv_ref[...] * t
    h1 = jnp.maximum(
        jnp.dot(a, W1_ref[...], preferred_element_type=jnp.float32)
        + b1_ref[...],
        0.0,
    )
    g2 = dinv_ref[...] * h1
    glo_ref[...] = g2[:, :16]
    ghi_ref[...] = g2[:, 16:]


def _mid(tp, g1, dinv, W1, b1r):
    return pl.pallas_call(
        _mid_body,
        grid=(NBLK,),
        in_specs=[
            pl.BlockSpec((NC, NB, 16), lambda i: (0, i, 0)),
            pl.BlockSpec((NB, 16), lambda i: (i, 0)),
            pl.BlockSpec((NB, 1), lambda i: (i, 0)),
            pl.BlockSpec((EMB, HID), lambda i: (0, 0)),
            pl.BlockSpec((1, HID), lambda i: (0, 0)),
        ],
        out_specs=[
            pl.BlockSpec((NB, 16), lambda i: (i, 0)),
            pl.BlockSpec((NB, 16), lambda i: (i, 0)),
        ],
        out_shape=[
            jax.ShapeDtypeStruct((N, 16), jnp.float32),
            jax.ShapeDtypeStruct((N, 16), jnp.float32),
        ],
    )(tp, g1, dinv, W1, b1r)


def _fin_body(tlo_ref, thi_ref, glo_ref, ghi_ref, dinv_ref, W2_ref, b2_ref,
              batch_ref, pool_ref):
    i = pl.program_id(0)
    dinv = dinv_ref[...]
    alo = dinv * (tlo_ref[0] + tlo_ref[1] + glo_ref[...])
    ahi = dinv * (thi_ref[0] + thi_ref[1] + ghi_ref[...])
    h2 = jnp.maximum(
        jnp.dot(alo, W2_ref[:16, :], preferred_element_type=jnp.float32)
        + jnp.dot(ahi, W2_ref[16:, :], preferred_element_type=jnp.float32)
        + b2_ref[...],
        0.0,
    )
    onehot = jnp.where(
        batch_ref[...] == lax.broadcasted_iota(jnp.int32, (NB, G), 1), 1.0, 0.0
    )
    part = lax.dot_general(
        onehot, h2, (((0,), (0,)), ((), ())),
        preferred_element_type=jnp.float32,
    )

    @pl.when(i == 0)
    def _():
        pool_ref[...] = jnp.zeros((G, HID), jnp.float32)

    pool_ref[...] += part


def _fin(tlo, thi, glo, ghi, dinv, W2, b2r, batch2d):
    return pl.pallas_call(
        _fin_body,
        grid=(NBLK,),
        in_specs=[
            pl.BlockSpec((NC, NB, 16), lambda i: (0, i, 0)),
            pl.BlockSpec((NC, NB, 16), lambda i: (0, i, 0)),
            pl.BlockSpec((NB, 16), lambda i: (i, 0)),
            pl.BlockSpec((NB, 16), lambda i: (i, 0)),
            pl.BlockSpec((NB, 1), lambda i: (i, 0)),
            pl.BlockSpec((HID, HID), lambda i: (0, 0)),
            pl.BlockSpec((1, HID), lambda i: (0, 0)),
            pl.BlockSpec((NB, 1), lambda i: (i, 0)),
        ],
        out_specs=pl.BlockSpec((G, HID), lambda i: (0, 0)),
        out_shape=jax.ShapeDtypeStruct((G, HID), jnp.float32),
    )(tlo, thi, glo, ghi, dinv, W2, b2r, batch2d)


def _head_body(pl_ref, cl_ref, pr_ref, cr_ref, Wout_ref, bout_ref, out_ref):
    ml = pl_ref[...] / jnp.maximum(cl_ref[...], 1.0)
    mr = pr_ref[...] / jnp.maximum(cr_ref[...], 1.0)
    out_ref[...] = (
        jnp.dot(ml, Wout_ref[:HID, :], preferred_element_type=jnp.float32)
        + jnp.dot(mr, Wout_ref[HID:, :], preferred_element_type=jnp.float32)
        + bout_ref[...]
    )


def _head(pool_l, cnt_l, pool_r, cnt_r, Wout, bout2d):
    return pl.pallas_call(
        _head_body,
        out_shape=jax.ShapeDtypeStruct((G, 1), jnp.float32),
    )(pool_l, cnt_l, pool_r, cnt_r, Wout, bout2d)


def _pad_edges_scat(src, dst):
    s2 = src.reshape(NW, E // NW)
    d2 = dst.reshape(NW, E // NW)
    padn = SC_EPT - E // NW
    spad = jnp.zeros((NW, padn), jnp.int32)
    dpad = jnp.broadcast_to(
        N + jnp.arange(NW, dtype=jnp.int32)[:, None], (NW, padn)
    )
    s3 = jnp.concatenate([s2, spad], axis=1).reshape(NW, SC_BLOCKS, 8, 128)
    d3 = jnp.concatenate([d2, dpad], axis=1).reshape(NW, SC_BLOCKS, 8, 128)
    return s3, d3


def kernel(lhs_x, lhs_edge_index, lhs_batch, rhs_x, rhs_edge_index, rhs_batch,
           emb, W1, b1, W2, b2, Wout, bout):
    src_l, dst_l = lhs_edge_index[0], lhs_edge_index[1]
    src_r, dst_r = rhs_edge_index[0], rhs_edge_index[1]

    padn = DEG_EPT - E // NS
    dpad = jnp.full((NS, padn), N, jnp.int32)
    dl = jnp.concatenate([dst_l.reshape(NS, E // NS), dpad], axis=1)
    dr = jnp.concatenate([dst_r.reshape(NS, E // NS), dpad], axis=1)
    deg_in = jnp.concatenate([dl, dr], axis=0).reshape(NW, DEG_BLOCKS, 1024)
    degp = _deg_kernel(deg_in)

    sl3, dl3 = _pad_edges_scat(src_l, dst_l)
    sr3, dr3 = _pad_edges_scat(src_r, dst_r)

    embp = jnp.pad(emb, ((0, 16 - emb.shape[0]), (0, 0)))
    b1r = b1.reshape(1, HID)
    b2r = b2.reshape(1, HID)
    batch_l2 = lhs_batch.reshape(N, 1)
    batch_r2 = rhs_batch.reshape(N, 1)

    dinv_l, g1_l, cnt_l = _prep(degp, lhs_x, batch_l2, embp, 0)
    dinv_r, g1_r, cnt_r = _prep(degp, rhs_x, batch_r2, embp, NS)

    t1_l = _scat_kernel(g1_l, sl3, dl3)
    t1_r = _scat_kernel(g1_r, sr3, dr3)

    g2lo_l, g2hi_l = _mid(t1_l, g1_l, dinv_l, W1, b1r)
    g2lo_r, g2hi_r = _mid(t1_r, g1_r, dinv_r, W1, b1r)

    t2lo_l = _scat_kernel(g2lo_l, sl3, dl3)
    t2hi_l = _scat_kernel(g2hi_l, sl3, dl3)
    t2lo_r = _scat_kernel(g2lo_r, sr3, dr3)
    t2hi_r = _scat_kernel(g2hi_r, sr3, dr3)

    pool_l = _fin(t2lo_l, t2hi_l, g2lo_l, g2hi_l, dinv_l, W2, b2r, batch_l2)
    pool_r = _fin(t2lo_r, t2hi_r, g2lo_r, g2hi_r, dinv_r, W2, b2r, batch_r2)

    return _head(
        pool_l, cnt_l.reshape(G, 1), pool_r, cnt_r.reshape(G, 1),
        Wout, bout.reshape(1, 1),
    )


# speedup vs baseline: 42.3463x; 1.3187x over previous
import functools

import jax
import jax.numpy as jnp
from jax import lax
from jax.experimental import pallas as pl
from jax.experimental.pallas import tpu as pltpu
from jax.experimental.pallas import tpu_sc as plsc

N = 100000
E = 1600000
G = 128
EMB = 16
HID = 32

NC = 2
NS = 16
NW = NC * NS

CH = E // 128
NP = 102400
NB = 2048
NBLK = NP // NB
ACC_CHUNKS = NP // 1024


def _deg_body(eil_ref, eir_ref, out_ref, acc, bufA, bufB, bufT, semA, semB):
    c = lax.axis_index("c")
    s = lax.axis_index("s")
    w = c * NS + s
    z16 = jnp.zeros((16,), jnp.float32)
    ones16 = jnp.ones((16,), jnp.float32)

    def zero_body(i, _):
        acc[pl.ds(i * 16, 16)] = z16
        return 0

    lax.fori_loop(0, NP // 16, zero_body, 0)

    start = 781 * s + jnp.minimum(s, 4)
    ntail = jnp.where(s < 4, 6, 5)

    def process(ei):
        def compute(buf):
            for j in range(8):
                for k in range(8):
                    idx = buf[j, pl.ds(k * 16, 16)]
                    plsc.addupdate_scatter(acc, [idx], ones16)

        pltpu.sync_copy(ei.at[1, pl.ds(start, 8)], bufA)
        pltpu.async_copy(ei.at[1, pl.ds(start + 8, 8)], bufB, semB)

        def body(i, _):
            compute(bufA)
            base = start + (2 * i + 2) * 8
            pltpu.async_copy(ei.at[1, pl.ds(base, 8)], bufA, semA)
            pltpu.make_async_copy(ei.at[1, pl.ds(0, 8)], bufB, semB).wait()
            compute(bufB)

            @pl.when(i < 47)
            def _():
                pltpu.async_copy(ei.at[1, pl.ds(base + 8, 8)], bufB, semB)

            pltpu.make_async_copy(ei.at[1, pl.ds(0, 8)], bufA, semA).wait()
            return 0

        lax.fori_loop(0, 48, body, 0)
        compute(bufA)

        def tbody(t, _):
            ch = start + 776 + t
            pltpu.sync_copy(ei.at[1, pl.ds(ch, 1)], bufT)
            for k in range(8):
                idx = bufT[0, pl.ds(k * 16, 16)]
                plsc.addupdate_scatter(acc, [idx], ones16)
            return 0

        lax.fori_loop(0, ntail, tbody, 0)

    @pl.when(c == 0)
    def _():
        process(eil_ref)

    @pl.when(c == 1)
    def _():
        process(eir_ref)

    pltpu.sync_copy(acc, out_ref.at[w])


_SC_PARAMS = pltpu.CompilerParams(
    needs_layout_passes=False, use_tc_tiling_on_sc=False
)


@functools.cache
def _deg_kernel_fn():
    return pl.kernel(
        _deg_body,
        out_type=jax.ShapeDtypeStruct((NW, NP), jnp.float32),
        compiler_params=_SC_PARAMS,
        mesh=plsc.VectorSubcoreMesh(
            core_axis_name="c", subcore_axis_name="s",
            num_cores=NC, num_subcores=NS,
        ),
        scratch_types=[
            pltpu.VMEM((NP,), jnp.float32),
            pltpu.VMEM((8, 128), jnp.int32),
            pltpu.VMEM((8, 128), jnp.int32),
            pltpu.VMEM((1, 128), jnp.int32),
            pltpu.SemaphoreType.DMA,
            pltpu.SemaphoreType.DMA,
        ],
    )


def _deg_kernel(eil3, eir3):
    return _deg_kernel_fn()(eil3, eir3)


def _scat_body(vtab_ref, ei_ref, out_ref, acc, sd, sdt, rows2,
               semi, semg, sems):
    c = lax.axis_index("c")
    s = lax.axis_index("s")
    w = c * NS + s
    z16 = jnp.zeros((16,), jnp.float32)
    start = 390 * w + jnp.minimum(w, 20)
    ntail = jnp.where(w < 20, 3, 2)

    def zrow(i, _):
        rows2[0, i, :] = z16
        return 0

    lax.fori_loop(0, 512, zrow, 0)
    for j in range(13):
        ch = s + NS * j

        @pl.when(ch < NP // 512)
        def _():
            pltpu.sync_copy(rows2.at[0], acc.at[pl.ds(ch * 512, 512)])

    plsc.subcore_barrier()

    def fire_gathers(slot, half):
        for j in range(4):
            pltpu.async_copy(
                vtab_ref.at[sd.at[slot, 0, j]],
                rows2.at[half, pl.ds(j * 128, 128)], semg,
            )

    pltpu.sync_copy(ei_ref.at[:, pl.ds(start, 4)], sd.at[0])
    pltpu.async_copy(ei_ref.at[:, pl.ds(start + 4, 4)], sd.at[1], semi)
    fire_gathers(0, 0)

    def body(b, _):
        par = lax.rem(b, 2)
        slot = lax.rem(b, 3)
        slot1 = lax.rem(b + 1, 3)
        slot2 = lax.rem(b + 2, 3)

        @pl.when(b > 0)
        def _():
            pltpu.make_async_copy(
                vtab_ref.at[pl.ds(0, 512)], rows2.at[1 - par], sems
            ).wait()

        @pl.when(b < 95)
        def _():
            pltpu.async_copy(
                ei_ref.at[:, pl.ds(start + (b + 2) * 4, 4)],
                sd.at[slot2], semi,
            )

        @pl.when(b < 96)
        def _():
            pltpu.make_async_copy(
                ei_ref.at[:, pl.ds(0, 4)], sd.at[slot1], semi
            ).wait()
            fire_gathers(slot1, 1 - par)

        pltpu.make_async_copy(
            vtab_ref.at[pl.ds(0, 512)], rows2.at[par], semg
        ).wait()
        for j in range(4):
            pltpu.async_copy(
                rows2.at[par, pl.ds(j * 128, 128)],
                acc.at[sd.at[slot, 1, j]], sems, add=True,
            )
        return 0

    lax.fori_loop(0, 97, body, 0)
    pltpu.make_async_copy(
        vtab_ref.at[pl.ds(0, 512)], rows2.at[0], sems
    ).wait()

    def tbody(t, _):
        ch = start + 388 + t
        pltpu.sync_copy(ei_ref.at[:, pl.ds(ch, 1)], sdt)
        pltpu.async_copy(
            vtab_ref.at[sdt.at[0, 0]], rows2.at[0, pl.ds(0, 128)], semg
        ).wait()
        pltpu.sync_copy(
            rows2.at[0, pl.ds(0, 128)], acc.at[sdt.at[1, 0]], add=True
        )
        return 0

    lax.fori_loop(0, ntail, tbody, 0)
    plsc.subcore_barrier()
    rpt = NP // NS
    pltpu.sync_copy(acc.at[pl.ds(s * rpt, rpt)],
                    out_ref.at[c, pl.ds(s * rpt, rpt)])


@functools.cache
def _scat_kernel_fn():
    return pl.kernel(
        _scat_body,
        out_type=jax.ShapeDtypeStruct((NC, NP, 16), jnp.float32),
        compiler_params=_SC_PARAMS,
        mesh=plsc.VectorSubcoreMesh(
            core_axis_name="c", subcore_axis_name="s",
            num_cores=NC, num_subcores=NS,
        ),
        scratch_types=[
            pltpu.VMEM_SHARED((NP, 16), jnp.float32),
            pltpu.VMEM((3, 2, 4, 128), jnp.int32),
            pltpu.VMEM((2, 1, 128), jnp.int32),
            pltpu.VMEM((2, 512, 16), jnp.float32),
            pltpu.SemaphoreType.DMA,
            pltpu.SemaphoreType.DMA,
            pltpu.SemaphoreType.DMA,
        ],
    )


def _scat_kernel(vtab, ei3):
    return _scat_kernel_fn()(vtab, ei3)


def _prep_body(degp_ref, x_ref, batch_ref, emb_ref, dinv_ref, g1_ref, cnt_ref,
               *, lo):
    i = pl.program_id(0)
    deg = jnp.sum(degp_ref[lo:lo + NS, :], axis=0) + 1.0
    dinv = lax.rsqrt(jnp.maximum(deg, 1.0))
    x = x_ref[...]
    h0 = jnp.zeros((NB, EMB), jnp.float32)
    for k in range(11):
        h0 = h0 + jnp.where(x == k, 1.0, 0.0) * emb_ref[k, :][None, :]
    dinv_ref[...] = dinv[:, None]
    g1_ref[...] = dinv[:, None] * h0
    onehot = jnp.where(
        batch_ref[...] == lax.broadcasted_iota(jnp.int32, (NB, G), 1), 1.0, 0.0
    )

    @pl.when(i == 0)
    def _():
        cnt_ref[...] = jnp.zeros((1, G), jnp.float32)

    cnt_ref[...] += jnp.sum(onehot, axis=0, keepdims=True)


def _prep(degp, x, batch2d, embp, lo):
    return pl.pallas_call(
        functools.partial(_prep_body, lo=lo),
        grid=(NBLK,),
        in_specs=[
            pl.BlockSpec((NW, NB), lambda i: (0, i)),
            pl.BlockSpec((NB, 1), lambda i: (i, 0)),
            pl.BlockSpec((NB, 1), lambda i: (i, 0)),
            pl.BlockSpec((16, EMB), lambda i: (0, 0)),
        ],
        out_specs=[
            pl.BlockSpec((NB, 1), lambda i: (i, 0)),
            pl.BlockSpec((NB, EMB), lambda i: (i, 0)),
            pl.BlockSpec((1, G), lambda i: (0, 0)),
        ],
        out_shape=[
            jax.ShapeDtypeStruct((NP, 1), jnp.float32),
            jax.ShapeDtypeStruct((NP, EMB), jnp.float32),
            jax.ShapeDtypeStruct((1, G), jnp.float32),
        ],
    )(degp, x, batch2d, embp)


def _mid_body(tp_ref, g1_ref, dinv_ref, W1_ref, b1_ref, glo_ref, ghi_ref):
    t = tp_ref[0] + tp_ref[1] + g1_ref[...]
    a = dinv_ref[...] * t
    h1 = jnp.maximum(
        jnp.dot(a, W1_ref[...], preferred_element_type=jnp.float32)
        + b1_ref[...],
        0.0,
    )
    g2 = dinv_ref[...] * h1
    glo_ref[...] = g2[:, :16]
    ghi_ref[...] = g2[:, 16:]


def _mid(tp, g1, dinv, W1, b1r):
    return pl.pallas_call(
        _mid_body,
        grid=(NBLK,),
        in_specs=[
            pl.BlockSpec((NC, NB, 16), lambda i: (0, i, 0)),
            pl.BlockSpec((NB, 16), lambda i: (i, 0)),
            pl.BlockSpec((NB, 1), lambda i: (i, 0)),
            pl.BlockSpec((EMB, HID), lambda i: (0, 0)),
            pl.BlockSpec((1, HID), lambda i: (0, 0)),
        ],
        out_specs=[
            pl.BlockSpec((NB, 16), lambda i: (i, 0)),
            pl.BlockSpec((NB, 16), lambda i: (i, 0)),
        ],
        out_shape=[
            jax.ShapeDtypeStruct((NP, 16), jnp.float32),
            jax.ShapeDtypeStruct((NP, 16), jnp.float32),
        ],
    )(tp, g1, dinv, W1, b1r)


def _fin_body(tlo_ref, thi_ref, glo_ref, ghi_ref, dinv_ref, W2_ref, b2_ref,
              batch_ref, pool_ref):
    i = pl.program_id(0)
    dinv = dinv_ref[...]
    alo = dinv * (tlo_ref[0] + tlo_ref[1] + glo_ref[...])
    ahi = dinv * (thi_ref[0] + thi_ref[1] + ghi_ref[...])
    h2 = jnp.maximum(
        jnp.dot(alo, W2_ref[:16, :], preferred_element_type=jnp.float32)
        + jnp.dot(ahi, W2_ref[16:, :], preferred_element_type=jnp.float32)
        + b2_ref[...],
        0.0,
    )
    onehot = jnp.where(
        batch_ref[...] == lax.broadcasted_iota(jnp.int32, (NB, G), 1), 1.0, 0.0
    )
    part = lax.dot_general(
        onehot, h2, (((0,), (0,)), ((), ())),
        preferred_element_type=jnp.float32,
    )

    @pl.when(i == 0)
    def _():
        pool_ref[...] = jnp.zeros((G, HID), jnp.float32)

    pool_ref[...] += part


def _fin(tlo, thi, glo, ghi, dinv, W2, b2r, batch2d):
    return pl.pallas_call(
        _fin_body,
        grid=(NBLK,),
        in_specs=[
            pl.BlockSpec((NC, NB, 16), lambda i: (0, i, 0)),
            pl.BlockSpec((NC, NB, 16), lambda i: (0, i, 0)),
            pl.BlockSpec((NB, 16), lambda i: (i, 0)),
            pl.BlockSpec((NB, 16), lambda i: (i, 0)),
            pl.BlockSpec((NB, 1), lambda i: (i, 0)),
            pl.BlockSpec((HID, HID), lambda i: (0, 0)),
            pl.BlockSpec((1, HID), lambda i: (0, 0)),
            pl.BlockSpec((NB, 1), lambda i: (i, 0)),
        ],
        out_specs=pl.BlockSpec((G, HID), lambda i: (0, 0)),
        out_shape=jax.ShapeDtypeStruct((G, HID), jnp.float32),
    )(tlo, thi, glo, ghi, dinv, W2, b2r, batch2d)


def _head_body(pl_ref, cl_ref, pr_ref, cr_ref, Wout_ref, bout_ref, out_ref):
    ml = pl_ref[...] / jnp.maximum(cl_ref[...], 1.0)
    mr = pr_ref[...] / jnp.maximum(cr_ref[...], 1.0)
    out_ref[...] = (
        jnp.dot(ml, Wout_ref[:HID, :], preferred_element_type=jnp.float32)
        + jnp.dot(mr, Wout_ref[HID:, :], preferred_element_type=jnp.float32)
        + bout_ref[...]
    )


def _head(pool_l, cnt_l, pool_r, cnt_r, Wout, bout2d):
    return pl.pallas_call(
        _head_body,
        out_shape=jax.ShapeDtypeStruct((G, 1), jnp.float32),
    )(pool_l, cnt_l, pool_r, cnt_r, Wout, bout2d)


def kernel(lhs_x, lhs_edge_index, lhs_batch, rhs_x, rhs_edge_index, rhs_batch,
           emb, W1, b1, W2, b2, Wout, bout):
    ei_l3 = lhs_edge_index.reshape(2, CH, 128)
    ei_r3 = rhs_edge_index.reshape(2, CH, 128)

    degp = _deg_kernel(ei_l3, ei_r3)

    embp = jnp.pad(emb, ((0, 16 - emb.shape[0]), (0, 0)))
    b1r = b1.reshape(1, HID)
    b2r = b2.reshape(1, HID)
    padn = NP - N
    x_l = jnp.pad(lhs_x, ((0, padn), (0, 0)))
    x_r = jnp.pad(rhs_x, ((0, padn), (0, 0)))
    batch_l2 = jnp.pad(lhs_batch.reshape(N, 1), ((0, padn), (0, 0)),
                       constant_values=G)
    batch_r2 = jnp.pad(rhs_batch.reshape(N, 1), ((0, padn), (0, 0)),
                       constant_values=G)

    dinv_l, g1_l, cnt_l = _prep(degp, x_l, batch_l2, embp, 0)
    dinv_r, g1_r, cnt_r = _prep(degp, x_r, batch_r2, embp, NS)

    t1_l = _scat_kernel(g1_l, ei_l3)
    t1_r = _scat_kernel(g1_r, ei_r3)

    g2lo_l, g2hi_l = _mid(t1_l, g1_l, dinv_l, W1, b1r)
    g2lo_r, g2hi_r = _mid(t1_r, g1_r, dinv_r, W1, b1r)

    t2lo_l = _scat_kernel(g2lo_l, ei_l3)
    t2hi_l = _scat_kernel(g2hi_l, ei_l3)
    t2lo_r = _scat_kernel(g2lo_r, ei_r3)
    t2hi_r = _scat_kernel(g2hi_r, ei_r3)

    pool_l = _fin(t2lo_l, t2hi_l, g2lo_l, g2hi_l, dinv_l, W2, b2r, batch_l2)
    pool_r = _fin(t2lo_r, t2hi_r, g2lo_r, g2hi_r, dinv_r, W2, b2r, batch_r2)

    return _head(
        pool_l, cnt_l.reshape(G, 1), pool_r, cnt_r.reshape(G, 1),
        Wout, bout.reshape(1, 1),
    )
